# Initial kernel scaffold; baseline (speedup 1.0000x reference)
import jax
import jax.numpy as jnp
from jax.experimental import pallas as pl


def kernel(data, org_edge_index, emb, lin_w, att_i, att_j, att_em_i, att_em_j, gnn_bias, bn1_g, bn1_b, bn2_g, bn2_b, out_w, out_b):
    raise NotImplementedError("write your pallas kernel here")


# speedup vs baseline: 104.3149x; 104.3149x over previous
import jax
import jax.numpy as jnp
from jax.experimental import pallas as pl

BATCH = 32
N = 512
DIM = 128
INPUT_DIM = 64
TOPK = 20
EPS = 1e-5

_INTERPRET = False


def _bf16_dot(a, b):
    return jax.lax.dot_general(
        a.astype(jnp.bfloat16), b.astype(jnp.bfloat16),
        (((1,), (0,)), ((), ())), preferred_element_type=jnp.float32)


def _graph_kernel(emb_ref, m_ref):
    emb = emb_ref[:]
    embb = emb.astype(jnp.bfloat16)
    gram = jax.lax.dot_general(embb, embb, (((1,), (1,)), ((), ())),
                               preferred_element_type=jnp.float32)
    nrm_col = jnp.sqrt(jnp.sum(emb * emb, axis=1, keepdims=True))
    nrm_row = nrm_col.T
    cos = gram / (nrm_col * nrm_row)
    lane = jax.lax.broadcasted_iota(jnp.int32, (N, N), 1)
    sub = jax.lax.broadcasted_iota(jnp.int32, (N, N), 0)

    def body(_, carry):
        c, m = carry
        rowmax = jnp.max(c, axis=1, keepdims=True)
        ismax = c == rowmax
        idx = jnp.min(jnp.where(ismax, lane, N), axis=1, keepdims=True)
        onehot = lane == idx
        m = jnp.where(onehot, 1.0, m)
        c = jnp.where(onehot, -2.0, c)
        return c, m

    _, m = jax.lax.fori_loop(0, TOPK, body,
                             (cos, jnp.zeros((N, N), jnp.float32)))
    m = jnp.where(sub == lane, 1.0, m)
    m_ref[:] = m


def _attn_kernel(data_ref, lin_w_ref, emb_ref, m_ref,
                 att_i_ref, att_j_ref, att_em_i_ref, att_em_j_ref, out_ref):
    xb = data_ref[0]
    xl = _bf16_dot(xb, lin_w_ref[:])
    emb = emb_ref[:]

    def score(mat, vec_ref):
        return jnp.sum(mat * vec_ref[:], axis=1, keepdims=True)

    ai_col = score(xl, att_i_ref) + score(emb, att_em_i_ref)
    aj_col = score(xl, att_j_ref) + score(emb, att_em_j_ref)
    alpha = ai_col + aj_col.T
    alpha = jnp.where(alpha > 0, alpha, 0.2 * alpha)
    msk = m_ref[:] > 0
    amax = jnp.max(jnp.where(msk, alpha, -1e30), axis=1, keepdims=True)
    e = jnp.where(msk, jnp.exp(alpha - amax), 0.0)
    den = jnp.sum(e, axis=1, keepdims=True)
    a = e / (den + 1e-16)
    out_ref[0] = jnp.dot(a, xl, preferred_element_type=jnp.float32,
                         precision=jax.lax.Precision.HIGHEST)


def _bnout_kernel(out_ref, emb_ref, gnn_bias_ref, bn1_g_ref, bn1_b_ref,
                  bn2_g_ref, bn2_b_ref, out_w_ref, res_ref):
    o = out_ref[:] + gnn_bias_ref[:]
    mu = jnp.mean(o, axis=0, keepdims=True)
    var = jnp.mean((o - mu) * (o - mu), axis=0, keepdims=True)
    o = (o - mu) / jnp.sqrt(var + EPS) * bn1_g_ref[:] + bn1_b_ref[:]
    o = jnp.maximum(o, 0.0)
    o = (o.reshape(BATCH, N, DIM) * emb_ref[:][None]).reshape(BATCH * N, DIM)
    mu2 = jnp.mean(o, axis=0, keepdims=True)
    var2 = jnp.mean((o - mu2) * (o - mu2), axis=0, keepdims=True)
    o = (o - mu2) / jnp.sqrt(var2 + EPS) * bn2_g_ref[:] + bn2_b_ref[:]
    o = jnp.maximum(o, 0.0)
    res_ref[:] = _bf16_dot(o, out_w_ref[:])


def kernel(data, org_edge_index, emb, lin_w, att_i, att_j, att_em_i, att_em_j,
           gnn_bias, bn1_g, bn1_b, bn2_g, bn2_b, out_w, out_b):
    del org_edge_index
    f32 = jnp.float32
    m = pl.pallas_call(
        _graph_kernel,
        out_shape=jax.ShapeDtypeStruct((N, N), f32),
        interpret=_INTERPRET,
    )(emb)

    att_i2 = att_i.reshape(1, DIM)
    att_j2 = att_j.reshape(1, DIM)
    att_em_i2 = att_em_i.reshape(1, DIM)
    att_em_j2 = att_em_j.reshape(1, DIM)

    full = lambda shape: pl.BlockSpec(shape, lambda b: (0,) * len(shape))
    out = pl.pallas_call(
        _attn_kernel,
        grid=(BATCH,),
        in_specs=[
            pl.BlockSpec((1, N, INPUT_DIM), lambda b: (b, 0, 0)),
            full((INPUT_DIM, DIM)),
            full((N, DIM)),
            full((N, N)),
            full((1, DIM)),
            full((1, DIM)),
            full((1, DIM)),
            full((1, DIM)),
        ],
        out_specs=pl.BlockSpec((1, N, DIM), lambda b: (b, 0, 0)),
        out_shape=jax.ShapeDtypeStruct((BATCH, N, DIM), f32),
        interpret=_INTERPRET,
    )(data, lin_w, emb, m, att_i2, att_j2, att_em_i2, att_em_j2)

    res = pl.pallas_call(
        _bnout_kernel,
        out_shape=jax.ShapeDtypeStruct((BATCH * N, 1), f32),
        interpret=_INTERPRET,
    )(out.reshape(BATCH * N, DIM), emb, gnn_bias.reshape(1, DIM),
      bn1_g.reshape(1, DIM), bn1_b.reshape(1, DIM),
      bn2_g.reshape(1, DIM), bn2_b.reshape(1, DIM), out_w)

    return (res + out_b).reshape(BATCH, N)


# speedup vs baseline: 119.6510x; 1.1470x over previous
import jax
import jax.numpy as jnp
from jax.experimental import pallas as pl

BATCH = 32
N = 512
DIM = 128
INPUT_DIM = 64
TOPK = 20
EPS = 1e-5

_INTERPRET = False


def _bf16_dot(a, b):
    return jax.lax.dot_general(
        a.astype(jnp.bfloat16), b.astype(jnp.bfloat16),
        (((1,), (0,)), ((), ())), preferred_element_type=jnp.float32)


def _graph_kernel(emb_ref, m_ref):
    emb = emb_ref[:]
    embb = emb.astype(jnp.bfloat16)
    gram = jax.lax.dot_general(embb, embb, (((1,), (1,)), ((), ())),
                               preferred_element_type=jnp.float32)
    nrm_col = jnp.sqrt(jnp.sum(emb * emb, axis=1, keepdims=True))
    nrm_row = nrm_col.T
    cos = gram / (nrm_col * nrm_row)
    lane = jax.lax.broadcasted_iota(jnp.int32, (N, N), 1)
    sub = jax.lax.broadcasted_iota(jnp.int32, (N, N), 0)

    def body(_, carry):
        c, m = carry
        rowmax = jnp.max(c, axis=1, keepdims=True)
        ismax = c == rowmax
        idx = jnp.min(jnp.where(ismax, lane, N), axis=1, keepdims=True)
        onehot = lane == idx
        m = jnp.where(onehot, 1.0, m)
        c = jnp.where(onehot, -2.0, c)
        return c, m

    diag = sub == lane
    m0 = jnp.where(diag, 1.0, 0.0)
    c0 = jnp.where(diag, -2.0, cos)
    _, m = jax.lax.fori_loop(0, TOPK - 1, body, (c0, m0))
    m_ref[:] = m


def _attn_kernel(data_ref, lin_w_ref, emb_ref, m_ref,
                 att_i_ref, att_j_ref, att_em_i_ref, att_em_j_ref, out_ref):
    xb = data_ref[0]
    xl = _bf16_dot(xb, lin_w_ref[:])
    emb = emb_ref[:]

    def score(mat, vec_ref):
        return jnp.sum(mat * vec_ref[:], axis=1, keepdims=True)

    ai_col = score(xl, att_i_ref) + score(emb, att_em_i_ref)
    aj_col = score(xl, att_j_ref) + score(emb, att_em_j_ref)
    alpha = ai_col + aj_col.T
    alpha = jnp.where(alpha > 0, alpha, 0.2 * alpha)
    msk = m_ref[:] > 0
    amax = jnp.max(jnp.where(msk, alpha, -1e30), axis=1, keepdims=True)
    e = jnp.where(msk, jnp.exp(alpha - amax), 0.0)
    den = jnp.sum(e, axis=1, keepdims=True)
    a = e / (den + 1e-16)
    a_hi = a.astype(jnp.bfloat16)
    a_lo = (a - a_hi.astype(jnp.float32)).astype(jnp.bfloat16)
    x_hi = xl.astype(jnp.bfloat16)
    x_lo = (xl - x_hi.astype(jnp.float32)).astype(jnp.bfloat16)
    dot = lambda p, q: jax.lax.dot_general(
        p, q, (((1,), (0,)), ((), ())), preferred_element_type=jnp.float32)
    out_ref[0] = dot(a_hi, x_hi) + (dot(a_hi, x_lo) + dot(a_lo, x_hi))


def _bnout_kernel(out_ref, emb_ref, gnn_bias_ref, bn1_g_ref, bn1_b_ref,
                  bn2_g_ref, bn2_b_ref, out_w_ref, res_ref):
    o = out_ref[:] + gnn_bias_ref[:]
    mu = jnp.mean(o, axis=0, keepdims=True)
    var = jnp.mean((o - mu) * (o - mu), axis=0, keepdims=True)
    o = (o - mu) / jnp.sqrt(var + EPS) * bn1_g_ref[:] + bn1_b_ref[:]
    o = jnp.maximum(o, 0.0)
    o = (o.reshape(BATCH, N, DIM) * emb_ref[:][None]).reshape(BATCH * N, DIM)
    mu2 = jnp.mean(o, axis=0, keepdims=True)
    var2 = jnp.mean((o - mu2) * (o - mu2), axis=0, keepdims=True)
    o = (o - mu2) / jnp.sqrt(var2 + EPS) * bn2_g_ref[:] + bn2_b_ref[:]
    o = jnp.maximum(o, 0.0)
    res_ref[:] = _bf16_dot(o, out_w_ref[:])


def kernel(data, org_edge_index, emb, lin_w, att_i, att_j, att_em_i, att_em_j,
           gnn_bias, bn1_g, bn1_b, bn2_g, bn2_b, out_w, out_b):
    del org_edge_index
    f32 = jnp.float32
    m = pl.pallas_call(
        _graph_kernel,
        out_shape=jax.ShapeDtypeStruct((N, N), f32),
        interpret=_INTERPRET,
    )(emb)

    att_i2 = att_i.reshape(1, DIM)
    att_j2 = att_j.reshape(1, DIM)
    att_em_i2 = att_em_i.reshape(1, DIM)
    att_em_j2 = att_em_j.reshape(1, DIM)

    full = lambda shape: pl.BlockSpec(shape, lambda b: (0,) * len(shape))
    out = pl.pallas_call(
        _attn_kernel,
        grid=(BATCH,),
        in_specs=[
            pl.BlockSpec((1, N, INPUT_DIM), lambda b: (b, 0, 0)),
            full((INPUT_DIM, DIM)),
            full((N, DIM)),
            full((N, N)),
            full((1, DIM)),
            full((1, DIM)),
            full((1, DIM)),
            full((1, DIM)),
        ],
        out_specs=pl.BlockSpec((1, N, DIM), lambda b: (b, 0, 0)),
        out_shape=jax.ShapeDtypeStruct((BATCH, N, DIM), f32),
        interpret=_INTERPRET,
    )(data, lin_w, emb, m, att_i2, att_j2, att_em_i2, att_em_j2)

    res = pl.pallas_call(
        _bnout_kernel,
        out_shape=jax.ShapeDtypeStruct((BATCH * N, 1), f32),
        interpret=_INTERPRET,
    )(out.reshape(BATCH * N, DIM), emb, gnn_bias.reshape(1, DIM),
      bn1_g.reshape(1, DIM), bn1_b.reshape(1, DIM),
      bn2_g.reshape(1, DIM), bn2_b.reshape(1, DIM), out_w)

    return (res + out_b).reshape(BATCH, N)


# speedup vs baseline: 119.9011x; 1.0021x over previous
import jax
import jax.numpy as jnp
from jax.experimental import pallas as pl

BATCH = 32
N = 512
DIM = 128
INPUT_DIM = 64
TOPK = 20
EPS = 1e-5
NEG = -1e30

_INTERPRET = False


def _bf16_dot(a, b):
    return jax.lax.dot_general(
        a.astype(jnp.bfloat16), b.astype(jnp.bfloat16),
        (((1,), (0,)), ((), ())), preferred_element_type=jnp.float32)


def _split3_dot(a, b):
    a_hi = a.astype(jnp.bfloat16)
    a_lo = (a - a_hi.astype(jnp.float32)).astype(jnp.bfloat16)
    b_hi = b.astype(jnp.bfloat16)
    b_lo = (b - b_hi.astype(jnp.float32)).astype(jnp.bfloat16)
    dot = lambda p, q: jax.lax.dot_general(
        p, q, (((1,), (0,)), ((), ())), preferred_element_type=jnp.float32)
    return dot(a_hi, b_hi) + (dot(a_hi, b_lo) + dot(a_lo, b_hi))


def _graph_kernel(emb_ref, att_em_i_ref, att_em_j_ref,
                  bias_ref, embi_ref, embj_ref):
    emb = emb_ref[:]
    embb = emb.astype(jnp.bfloat16)
    gram = jax.lax.dot_general(embb, embb, (((1,), (1,)), ((), ())),
                               preferred_element_type=jnp.float32)
    nrm_col = jnp.sqrt(jnp.sum(emb * emb, axis=1, keepdims=True))
    nrm_row = nrm_col.T
    cos = gram / (nrm_col * nrm_row)
    lane = jax.lax.broadcasted_iota(jnp.int32, (N, N), 1)
    sub = jax.lax.broadcasted_iota(jnp.int32, (N, N), 0)

    def body(_, carry):
        c, m = carry
        rowmax = jnp.max(c, axis=1, keepdims=True)
        ismax = c == rowmax
        idx = jnp.min(jnp.where(ismax, lane, N), axis=1, keepdims=True)
        onehot = lane == idx
        m = jnp.where(onehot, 0.0, m)
        c = jnp.where(onehot, -2.0, c)
        return c, m

    diag = sub == lane
    m0 = jnp.where(diag, 0.0, NEG)
    c0 = jnp.where(diag, -2.0, cos)
    _, m = jax.lax.fori_loop(0, TOPK - 1, body, (c0, m0))
    bias_ref[:] = m
    embi_ref[:] = jnp.sum(emb * att_em_i_ref[:], axis=1, keepdims=True)
    embj_col = jnp.sum(emb * att_em_j_ref[:], axis=1, keepdims=True)
    embj_ref[:] = embj_col.T


def _attn_kernel(data_ref, lin_w_ref, bias_ref, embi_ref, embj_ref,
                 att_i_ref, att_j_ref, out_ref):
    xb = data_ref[0]
    xl = _bf16_dot(xb, lin_w_ref[:])
    ai_col = jnp.sum(xl * att_i_ref[:], axis=1, keepdims=True) + embi_ref[:]
    aj_col = jnp.sum(xl * att_j_ref[:], axis=1, keepdims=True)
    aj_row = aj_col.T + embj_ref[:]
    alpha = ai_col + aj_row
    alpha = jnp.maximum(alpha, 0.2 * alpha) + bias_ref[:]
    amax = jnp.max(alpha, axis=1, keepdims=True)
    e = jnp.exp(alpha - amax)
    den = jnp.sum(e, axis=1, keepdims=True)
    agg = _split3_dot(e, xl)
    out_ref[0] = agg * (1.0 / (den + 1e-16))


def _bnout_kernel(out_ref, emb_ref, gnn_bias_ref, bn1_g_ref, bn1_b_ref,
                  bn2_g_ref, bn2_b_ref, out_w_ref, res_ref):
    o = out_ref[:] + gnn_bias_ref[:]
    inv = 1.0 / (BATCH * N)
    mu = jnp.sum(o, axis=0, keepdims=True) * inv
    var = jnp.sum(o * o, axis=0, keepdims=True) * inv - mu * mu
    o = (o - mu) / jnp.sqrt(var + EPS) * bn1_g_ref[:] + bn1_b_ref[:]
    o = jnp.maximum(o, 0.0)
    o = (o.reshape(BATCH, N, DIM) * emb_ref[:][None]).reshape(BATCH * N, DIM)
    mu2 = jnp.sum(o, axis=0, keepdims=True) * inv
    var2 = jnp.sum(o * o, axis=0, keepdims=True) * inv - mu2 * mu2
    o = (o - mu2) / jnp.sqrt(var2 + EPS) * bn2_g_ref[:] + bn2_b_ref[:]
    o = jnp.maximum(o, 0.0)
    res_ref[:] = _bf16_dot(o, out_w_ref[:])


def kernel(data, org_edge_index, emb, lin_w, att_i, att_j, att_em_i, att_em_j,
           gnn_bias, bn1_g, bn1_b, bn2_g, bn2_b, out_w, out_b):
    del org_edge_index
    f32 = jnp.float32
    bias, embi, embj = pl.pallas_call(
        _graph_kernel,
        out_shape=(jax.ShapeDtypeStruct((N, N), f32),
                   jax.ShapeDtypeStruct((N, 1), f32),
                   jax.ShapeDtypeStruct((1, N), f32)),
        interpret=_INTERPRET,
    )(emb, att_em_i.reshape(1, DIM), att_em_j.reshape(1, DIM))

    full = lambda shape: pl.BlockSpec(shape, lambda b: (0,) * len(shape))
    out = pl.pallas_call(
        _attn_kernel,
        grid=(BATCH,),
        in_specs=[
            pl.BlockSpec((1, N, INPUT_DIM), lambda b: (b, 0, 0)),
            full((INPUT_DIM, DIM)),
            full((N, N)),
            full((N, 1)),
            full((1, N)),
            full((1, DIM)),
            full((1, DIM)),
        ],
        out_specs=pl.BlockSpec((1, N, DIM), lambda b: (b, 0, 0)),
        out_shape=jax.ShapeDtypeStruct((BATCH, N, DIM), f32),
        interpret=_INTERPRET,
    )(data, lin_w, bias, embi, embj,
      att_i.reshape(1, DIM), att_j.reshape(1, DIM))

    res = pl.pallas_call(
        _bnout_kernel,
        out_shape=jax.ShapeDtypeStruct((BATCH * N, 1), f32),
        interpret=_INTERPRET,
    )(out.reshape(BATCH * N, DIM), emb, gnn_bias.reshape(1, DIM),
      bn1_g.reshape(1, DIM), bn1_b.reshape(1, DIM),
      bn2_g.reshape(1, DIM), bn2_b.reshape(1, DIM), out_w)

    return (res + out_b).reshape(BATCH, N)


# speedup vs baseline: 135.3634x; 1.1290x over previous
import jax
import jax.numpy as jnp
from jax.experimental import pallas as pl

BATCH = 32
N = 512
DIM = 128
INPUT_DIM = 64
TOPK = 20
EPS = 1e-5
NEG = -1e30
BSUB = 4

_INTERPRET = False


def _bf16_dot(a, b):
    return jax.lax.dot_general(
        a.astype(jnp.bfloat16), b.astype(jnp.bfloat16),
        (((1,), (0,)), ((), ())), preferred_element_type=jnp.float32)


def _split3_dot(a, b):
    a_hi = a.astype(jnp.bfloat16)
    a_lo = (a - a_hi.astype(jnp.float32)).astype(jnp.bfloat16)
    b_hi = b.astype(jnp.bfloat16)
    b_lo = (b - b_hi.astype(jnp.float32)).astype(jnp.bfloat16)
    dot = lambda p, q: jax.lax.dot_general(
        p, q, (((1,), (0,)), ((), ())), preferred_element_type=jnp.float32)
    return dot(a_hi, b_hi) + (dot(a_hi, b_lo) + dot(a_lo, b_hi))


def _graph_kernel(emb_ref, att_em_i_ref, att_em_j_ref,
                  bias_ref, embi_ref, embj_ref):
    emb = emb_ref[:]
    embb = emb.astype(jnp.bfloat16)
    gram = jax.lax.dot_general(embb, embb, (((1,), (1,)), ((), ())),
                               preferred_element_type=jnp.float32)
    nrm_col = jnp.sqrt(jnp.sum(emb * emb, axis=1, keepdims=True))
    nrm_row = nrm_col.T
    cos = gram / (nrm_col * nrm_row)
    lane = jax.lax.broadcasted_iota(jnp.int32, (N, N), 1)
    sub = jax.lax.broadcasted_iota(jnp.int32, (N, N), 0)

    def body(_, carry):
        c, m = carry
        rowmax = jnp.max(c, axis=1, keepdims=True)
        ismax = c == rowmax
        idx = jnp.min(jnp.where(ismax, lane, N), axis=1, keepdims=True)
        onehot = lane == idx
        m = jnp.where(onehot, 0.0, m)
        c = jnp.where(onehot, -2.0, c)
        return c, m

    diag = sub == lane
    m0 = jnp.where(diag, 0.0, NEG)
    c0 = jnp.where(diag, -2.0, cos)
    _, m = jax.lax.fori_loop(0, TOPK - 1, body, (c0, m0))
    bias_ref[:] = m
    embi_ref[:] = jnp.sum(emb * att_em_i_ref[:], axis=1, keepdims=True)
    embj_col = jnp.sum(emb * att_em_j_ref[:], axis=1, keepdims=True)
    embj_ref[:] = embj_col.T


def _attn_kernel(data_ref, lin_w_ref, bias_ref, embi_ref, embj_ref,
                 att_i_ref, att_j_ref, out_ref):
    for i in range(BSUB):
        xb = data_ref[i]
        xl = _bf16_dot(xb, lin_w_ref[:])
        ai_col = jnp.sum(xl * att_i_ref[:], axis=1, keepdims=True) + embi_ref[:]
        aj_col = jnp.sum(xl * att_j_ref[:], axis=1, keepdims=True)
        aj_row = aj_col.T + embj_ref[:]
        alpha = ai_col + aj_row
        alpha = jnp.maximum(alpha, 0.2 * alpha) + bias_ref[:]
        amax = jnp.max(alpha, axis=1, keepdims=True)
        e = jnp.exp(alpha - amax)
        den = jnp.sum(e, axis=1, keepdims=True)
        agg = _split3_dot(e, xl)
        out_ref[i] = agg * (1.0 / (den + 1e-16))


def _bnout_kernel(out_ref, emb_ref, gnn_bias_ref, bn1_g_ref, bn1_b_ref,
                  bn2_g_ref, bn2_b_ref, out_w_ref, res_ref):
    o = out_ref[:] + gnn_bias_ref[:]
    inv = 1.0 / (BATCH * N)
    mu = jnp.sum(o, axis=0, keepdims=True) * inv
    var = jnp.sum(o * o, axis=0, keepdims=True) * inv - mu * mu
    o = (o - mu) / jnp.sqrt(var + EPS) * bn1_g_ref[:] + bn1_b_ref[:]
    o = jnp.maximum(o, 0.0)
    o = (o.reshape(BATCH, N, DIM) * emb_ref[:][None]).reshape(BATCH * N, DIM)
    mu2 = jnp.sum(o, axis=0, keepdims=True) * inv
    var2 = jnp.sum(o * o, axis=0, keepdims=True) * inv - mu2 * mu2
    o = (o - mu2) / jnp.sqrt(var2 + EPS) * bn2_g_ref[:] + bn2_b_ref[:]
    o = jnp.maximum(o, 0.0)
    res_ref[:] = _bf16_dot(o, out_w_ref[:])


def kernel(data, org_edge_index, emb, lin_w, att_i, att_j, att_em_i, att_em_j,
           gnn_bias, bn1_g, bn1_b, bn2_g, bn2_b, out_w, out_b):
    del org_edge_index
    f32 = jnp.float32
    bias, embi, embj = pl.pallas_call(
        _graph_kernel,
        out_shape=(jax.ShapeDtypeStruct((N, N), f32),
                   jax.ShapeDtypeStruct((N, 1), f32),
                   jax.ShapeDtypeStruct((1, N), f32)),
        interpret=_INTERPRET,
    )(emb, att_em_i.reshape(1, DIM), att_em_j.reshape(1, DIM))

    full = lambda shape: pl.BlockSpec(shape, lambda b: (0,) * len(shape))
    out = pl.pallas_call(
        _attn_kernel,
        grid=(BATCH // BSUB,),
        in_specs=[
            pl.BlockSpec((BSUB, N, INPUT_DIM), lambda b: (b, 0, 0)),
            full((INPUT_DIM, DIM)),
            full((N, N)),
            full((N, 1)),
            full((1, N)),
            full((1, DIM)),
            full((1, DIM)),
        ],
        out_specs=pl.BlockSpec((BSUB, N, DIM), lambda b: (b, 0, 0)),
        out_shape=jax.ShapeDtypeStruct((BATCH, N, DIM), f32),
        interpret=_INTERPRET,
    )(data, lin_w, bias, embi, embj,
      att_i.reshape(1, DIM), att_j.reshape(1, DIM))

    res = pl.pallas_call(
        _bnout_kernel,
        out_shape=jax.ShapeDtypeStruct((BATCH * N, 1), f32),
        interpret=_INTERPRET,
    )(out.reshape(BATCH * N, DIM), emb, gnn_bias.reshape(1, DIM),
      bn1_g.reshape(1, DIM), bn1_b.reshape(1, DIM),
      bn2_g.reshape(1, DIM), bn2_b.reshape(1, DIM), out_w)

    return (res + out_b).reshape(BATCH, N)
